# baseline (speedup 1.0000x reference)
import functools

import jax
import jax.numpy as jnp
from jax import lax
from jax.experimental import pallas as pl
from jax.experimental.pallas import tpu as pltpu
from jax.experimental.pallas import tpu_sc as plsc

N_NODES = 10000
N_PAD = 10240
D = 128
TILES = 32
CHUNK = 128
NEG = 0.2


def _prep_body(x_ref, w_ref, as_ref, ad_ref, h0_ref, h1_ref, als_ref,
               ald_ref, cs_ref, cd_ref):
    i = pl.program_id(0)
    h = jnp.dot(x_ref[...], w_ref[...], preferred_element_type=jnp.float32)
    h0_ref[...] = h[:, : D // 2]
    h1_ref[...] = h[:, D // 2:]
    als = jnp.sum(h * as_ref[...], axis=1, keepdims=True)
    ald = jnp.sum(h * ad_ref[...], axis=1, keepdims=True)
    als_ref[...] = als
    ald_ref[...] = ald

    @pl.when(i == 0)
    def _():
        cs_ref[...] = jnp.full((1, 16), -jnp.inf, jnp.float32)
        cd_ref[...] = jnp.full((1, 16), -jnp.inf, jnp.float32)

    cs_ref[...] = jnp.maximum(cs_ref[...], jnp.max(als))
    cd_ref[...] = jnp.maximum(cd_ref[...], jnp.max(ald))


def _tc_prep(x, W, att_src, att_dst):
    n = x.shape[0]
    blk = 1000
    grid = n // blk
    return pl.pallas_call(
        _prep_body,
        grid=(grid,),
        in_specs=[
            pl.BlockSpec((blk, D), lambda i: (i, 0)),
            pl.BlockSpec((D, D), lambda i: (0, 0)),
            pl.BlockSpec((1, D), lambda i: (0, 0)),
            pl.BlockSpec((1, D), lambda i: (0, 0)),
        ],
        out_specs=[
            pl.BlockSpec((blk, D // 2), lambda i: (i, 0)),
            pl.BlockSpec((blk, D // 2), lambda i: (i, 0)),
            pl.BlockSpec((blk, 1), lambda i: (i, 0)),
            pl.BlockSpec((blk, 1), lambda i: (i, 0)),
            pl.BlockSpec((1, 16), lambda i: (0, 0)),
            pl.BlockSpec((1, 16), lambda i: (0, 0)),
        ],
        out_shape=[
            jax.ShapeDtypeStruct((n, D // 2), jnp.float32),
            jax.ShapeDtypeStruct((n, D // 2), jnp.float32),
            jax.ShapeDtypeStruct((n, 1), jnp.float32),
            jax.ShapeDtypeStruct((n, 1), jnp.float32),
            jax.ShapeDtypeStruct((1, 16), jnp.float32),
            jax.ShapeDtypeStruct((1, 16), jnp.float32),
        ],
    )(x, W, att_src.reshape(1, D), att_dst.reshape(1, D))


def _make_sc_edge(n_chunks, e_tot):
    mesh = plsc.VectorSubcoreMesh(core_axis_name="c", subcore_axis_name="s")
    e_per_tile = n_chunks * CHUNK
    stripe = N_PAD // 16
    DH = D // 2

    @functools.partial(
        pl.kernel,
        out_type=[
            jax.ShapeDtypeStruct((N_PAD, D), jnp.float32),
        ],
        mesh=mesh,
        scratch_types=[
            pltpu.VMEM((N_NODES,), jnp.float32),
            pltpu.VMEM((N_NODES,), jnp.float32),
            pltpu.VMEM((N_PAD // 16,), jnp.float32),
            pltpu.VMEM((D,), jnp.float32),
            pltpu.VMEM((n_chunks // 2, CHUNK), jnp.int32),
            pltpu.VMEM((n_chunks // 2, CHUNK), jnp.int32),
            pltpu.VMEM((4 * CHUNK,), jnp.float32),
            pltpu.VMEM((4 * CHUNK, DH), jnp.bfloat16),
            pltpu.VMEM((2 * CHUNK, DH), jnp.float32),
            pltpu.VMEM((16,), jnp.float32),
            pltpu.VMEM((16,), jnp.float32),
            pltpu.VMEM_SHARED((N_PAD, DH), jnp.float32),
            pltpu.VMEM_SHARED((N_PAD,), jnp.float32),
            pltpu.SemaphoreType.DMA,
            pltpu.SemaphoreType.DMA,
            pltpu.SemaphoreType.DMA,
        ],
        compiler_params=pltpu.CompilerParams(
            needs_layout_passes=False, use_tc_tiling_on_sc=False),
    )
    def sc_edge(h0_hbm, h1_hbm, as_hbm, ad_hbm, cs_hbm, cd_hbm, src_hbm,
                dst_hbm, bias_hbm, out_hbm,
                as_v, ad_v, den_v, bias_v, src_v, dst_v, w_v, rows_bf, rows_f,
                cs_v, cd_v, acc_sh, den_sh, gsem, ssem, dsem):
        cid = lax.axis_index("c")
        sid = lax.axis_index("s")
        row0 = sid * stripe
        zero16 = jnp.zeros((16,), jnp.float32)

        def _zr(r, carry):
            for k in range(DH // 16):
                rows_f[r, pl.ds(k * 16, 16)] = zero16
            return carry
        lax.fori_loop(0, CHUNK, _zr, 0)
        for j in range(8):
            w_v[pl.ds(j * 16, 16)] = zero16
        for b in range(stripe // CHUNK):
            pltpu.sync_copy(rows_f.at[pl.ds(0, CHUNK)],
                            acc_sh.at[pl.ds(row0 + b * CHUNK, CHUNK)])
        for b in range(stripe // CHUNK):
            pltpu.sync_copy(w_v.at[pl.ds(0, CHUNK)],
                            den_sh.at[pl.ds(row0 + b * CHUNK, CHUNK)])

        pltpu.sync_copy(as_hbm, as_v)
        pltpu.sync_copy(ad_hbm, ad_v)
        pltpu.sync_copy(cs_hbm, cs_v)
        pltpu.sync_copy(cd_hbm, cd_v)
        plsc.subcore_barrier()

        cm = cs_v[...] + cd_v[...]
        vc = jnp.where(cm > 0, cm, NEG * cm)
        iota16 = lax.iota(jnp.int32, 16)
        base = sid * e_per_tile
        NB = 4

        def rows_buf(b):
            return rows_bf.at[pl.ds(b * CHUNK, CHUNK)]

        def rowsf_buf(b):
            return rows_f.at[pl.ds(b * CHUNK, CHUNK)]

        def w_buf(b):
            return w_v.at[pl.ds(b * CHUNK, CHUNK)]

        def start_gather(ch, b):
            @pl.when(cid == 0)
            def _():
                pltpu.async_copy(h0_hbm.at[src_v.at[ch]], rows_buf(b), gsem)

            @pl.when(cid == 1)
            def _():
                pltpu.async_copy(h1_hbm.at[src_v.at[ch]], rows_buf(b), gsem)

        def wait_gather(b):
            pltpu.make_async_copy(h0_hbm.at[src_v.at[0]], rows_buf(b), gsem).wait()

        def wait_scatter(b):
            pltpu.make_async_copy(rowsf_buf(0), acc_sh.at[dst_v.at[0]], ssem).wait()
            pltpu.make_async_copy(w_buf(b), den_sh.at[dst_v.at[0]], dsem).wait()

        hc = n_chunks // 2

        for p in range(2):
            pltpu.sync_copy(src_hbm.at[sid, pl.ds(p * hc, hc)], src_v)
            pltpu.sync_copy(dst_hbm.at[sid, pl.ds(p * hc, hc)], dst_v)
            base_p = base + p * hc * CHUNK
            start_gather(0, 0)
            start_gather(1, 1)

            def chunk_body(ch, carry, base_p=base_p):
                b = lax.rem(ch, NB)
                pb = lax.rem(ch + 2, NB)
                wb = b * CHUNK

                @pl.when(ch >= 2)
                def _():
                    wait_scatter(pb)

                @pl.when(ch + 2 < hc)
                def _():
                    start_gather(ch + 2, pb)

                for j in range(8):
                    sv = src_v[ch, pl.ds(j * 16, 16)]
                    dv = dst_v[ch, pl.ds(j * 16, 16)]
                    a = plsc.load_gather(as_v, [sv]) + plsc.load_gather(ad_v, [dv])
                    a = jnp.where(a > 0, a, NEG * a)
                    w = jnp.exp(a - vc)
                    eid = base_p + ch * CHUNK + j * 16 + iota16
                    w = jnp.where(eid < e_tot, w, 0.0)
                    w_v[pl.ds(wb + j * 16, 16)] = w

                pltpu.async_copy(w_buf(b), den_sh.at[dst_v.at[ch]], dsem,
                                 add=True)

                wait_gather(b)
                parity = lax.rem(ch, 2)
                fb = parity * CHUNK

                @plsc.parallel_loop(0, CHUNK, unroll=4)
                def _(e):
                    we = plsc.load_gather(w_v, [jnp.broadcast_to(wb + e, (16,))])
                    r = wb + e
                    rf = fb + e
                    for g in range(DH // 32):
                        v32 = rows_bf[r, pl.ds(g * 32, 32)]
                        lo, hi = plsc.unpack(
                            v32, format=plsc.PackFormat.INTERLEAVED,
                            preferred_element_type=jnp.float32)
                        rows_f[rf, pl.ds(g * 32, 16)] = lo * we
                        rows_f[rf, pl.ds(g * 32 + 16, 16)] = hi * we

                pltpu.async_copy(rowsf_buf(parity), acc_sh.at[dst_v.at[ch]],
                                 ssem, add=True)
                return carry

            lax.fori_loop(0, hc, chunk_body, 0)
            wait_scatter((hc - 2) % NB)
            wait_scatter((hc - 1) % NB)
        plsc.subcore_barrier()

        pltpu.sync_copy(den_sh.at[pl.ds(row0, stripe)], den_v)
        pltpu.sync_copy(bias_hbm, bias_v)
        col0 = cid * DH
        for i in range(stripe // 16):
            sl = pl.ds(i * 16, 16)
            den_v[sl] = 1.0 / (den_v[sl] + 1e-16)
        out_row0 = row0
        off = 0
        for sz in (CHUNK, CHUNK, CHUNK, CHUNK, CHUNK):
            pltpu.sync_copy(acc_sh.at[pl.ds(out_row0 + off, sz)],
                            rows_f.at[pl.ds(0, sz)])
            _off = off

            @plsc.parallel_loop(0, sz, unroll=4)
            def _(r):
                d = plsc.load_gather(
                    den_v, [jnp.broadcast_to(_off + r, (16,))])
                for k in range(DH // 16):
                    sl = pl.ds(k * 16, 16)
                    bv = bias_v[pl.ds(col0 + k * 16, 16)]
                    rows_f[r, sl] = jnp.maximum(rows_f[r, sl] * d + bv, 0.0)
            pltpu.sync_copy(rows_f.at[pl.ds(0, sz)],
                            out_hbm.at[pl.ds(out_row0 + off, sz),
                                       pl.ds(col0, DH)])
            off += sz

    return sc_edge


def _interleave_bf16(hh):
    n = hh.shape[0]
    t = hh.reshape(n, 2, 2, 16).transpose(0, 1, 3, 2).reshape(n, 64)
    return t.astype(jnp.bfloat16)


@jax.jit
def kernel(x, edge_index, W, att_src, att_dst, bias):
    n = x.shape[0]
    e = edge_index.shape[1]
    e_tot = e + n

    h0, h1, als, ald, cs, cd = _tc_prep(x, W, att_src, att_dst)
    a_src = als.reshape(n)
    a_dst = ald.reshape(n)

    ei = edge_index.astype(jnp.int32)
    loops = jnp.arange(n, dtype=jnp.int32)
    src = jnp.concatenate([ei[0], loops])
    dst = jnp.concatenate([ei[1], loops])
    n_sub = 16
    per_round = n_sub * CHUNK
    n_chunks = (e_tot + per_round - 1) // per_round
    e_pad = n_chunks * per_round
    src = jnp.pad(src, (0, e_pad - e_tot)).reshape(n_sub, n_chunks, CHUNK)
    dst = jnp.pad(dst, (0, e_pad - e_tot)).reshape(n_sub, n_chunks, CHUNK)

    sc_edge = _make_sc_edge(n_chunks, e_tot)
    (out,) = sc_edge(_interleave_bf16(h0), _interleave_bf16(h1),
                     a_src, a_dst, cs.reshape(16), cd.reshape(16),
                     src, dst, bias)
    return out[:n]


# speedup vs baseline: 1.0173x; 1.0173x over previous
import functools

import jax
import jax.numpy as jnp
from jax import lax
from jax.experimental import pallas as pl
from jax.experimental.pallas import tpu as pltpu
from jax.experimental.pallas import tpu_sc as plsc

N_NODES = 10000
N_PAD = 10240
D = 128
TILES = 32
CHUNK = 128
NEG = 0.2


def _prep_body(x_ref, w_ref, as_ref, ad_ref, h0_ref, h1_ref, als_ref, ald_ref):
    h = jnp.dot(x_ref[...], w_ref[...], preferred_element_type=jnp.float32)
    h0_ref[...] = h[:, : D // 2]
    h1_ref[...] = h[:, D // 2:]
    als_ref[...] = jnp.sum(h * as_ref[...], axis=1, keepdims=True)
    ald_ref[...] = jnp.sum(h * ad_ref[...], axis=1, keepdims=True)


def _tc_prep(x, W, att_src, att_dst):
    n = x.shape[0]
    blk = 1000
    grid = n // blk
    return pl.pallas_call(
        _prep_body,
        grid=(grid,),
        in_specs=[
            pl.BlockSpec((blk, D), lambda i: (i, 0)),
            pl.BlockSpec((D, D), lambda i: (0, 0)),
            pl.BlockSpec((1, D), lambda i: (0, 0)),
            pl.BlockSpec((1, D), lambda i: (0, 0)),
        ],
        out_specs=[
            pl.BlockSpec((blk, D // 2), lambda i: (i, 0)),
            pl.BlockSpec((blk, D // 2), lambda i: (i, 0)),
            pl.BlockSpec((blk, 1), lambda i: (i, 0)),
            pl.BlockSpec((blk, 1), lambda i: (i, 0)),
        ],
        out_shape=[
            jax.ShapeDtypeStruct((n, D // 2), jnp.float32),
            jax.ShapeDtypeStruct((n, D // 2), jnp.float32),
            jax.ShapeDtypeStruct((n, 1), jnp.float32),
            jax.ShapeDtypeStruct((n, 1), jnp.float32),
        ],
    )(x, W, att_src.reshape(1, D), att_dst.reshape(1, D))


def _make_sc_edge(n_chunks, e_tot):
    mesh = plsc.VectorSubcoreMesh(core_axis_name="c", subcore_axis_name="s")
    e_per_tile = n_chunks * CHUNK
    stripe = N_PAD // 16
    DH = D // 2

    @functools.partial(
        pl.kernel,
        out_type=[
            jax.ShapeDtypeStruct((N_PAD, D), jnp.float32),
        ],
        mesh=mesh,
        scratch_types=[
            pltpu.VMEM((N_NODES,), jnp.float32),
            pltpu.VMEM((N_NODES,), jnp.float32),
            pltpu.VMEM((N_PAD // 16,), jnp.float32),
            pltpu.VMEM((D,), jnp.float32),
            pltpu.VMEM((n_chunks // 2, CHUNK), jnp.int32),
            pltpu.VMEM((n_chunks // 2, CHUNK), jnp.int32),
            pltpu.VMEM((4 * CHUNK,), jnp.float32),
            pltpu.VMEM((4 * CHUNK, DH), jnp.bfloat16),
            pltpu.VMEM((2 * CHUNK, DH), jnp.float32),
            pltpu.VMEM((16,), jnp.float32),
            pltpu.VMEM_SHARED((N_PAD, DH), jnp.float32),
            pltpu.VMEM_SHARED((N_PAD,), jnp.float32),
            pltpu.SemaphoreType.DMA,
            pltpu.SemaphoreType.DMA,
            pltpu.SemaphoreType.DMA,
        ],
        compiler_params=pltpu.CompilerParams(
            needs_layout_passes=False, use_tc_tiling_on_sc=False),
    )
    def sc_edge(h0_hbm, h1_hbm, as_hbm, ad_hbm, c_hbm, src_hbm, dst_hbm,
                bias_hbm, out_hbm,
                as_v, ad_v, den_v, bias_v, src_v, dst_v, w_v, rows_bf, rows_f,
                c_v, acc_sh, den_sh, gsem, ssem, dsem):
        cid = lax.axis_index("c")
        sid = lax.axis_index("s")
        row0 = sid * stripe
        zero16 = jnp.zeros((16,), jnp.float32)

        def _zr(r, carry):
            for k in range(DH // 16):
                rows_f[r, pl.ds(k * 16, 16)] = zero16
            return carry
        lax.fori_loop(0, CHUNK, _zr, 0)
        for j in range(8):
            w_v[pl.ds(j * 16, 16)] = zero16
        for b in range(stripe // CHUNK):
            pltpu.sync_copy(rows_f.at[pl.ds(0, CHUNK)],
                            acc_sh.at[pl.ds(row0 + b * CHUNK, CHUNK)])
        for b in range(stripe // CHUNK):
            pltpu.sync_copy(w_v.at[pl.ds(0, CHUNK)],
                            den_sh.at[pl.ds(row0 + b * CHUNK, CHUNK)])

        pltpu.sync_copy(as_hbm, as_v)
        pltpu.sync_copy(ad_hbm, ad_v)
        pltpu.sync_copy(c_hbm, c_v)
        plsc.subcore_barrier()

        vc = c_v[...]
        iota16 = lax.iota(jnp.int32, 16)
        base = sid * e_per_tile
        NB = 4

        def rows_buf(b):
            return rows_bf.at[pl.ds(b * CHUNK, CHUNK)]

        def rowsf_buf(b):
            return rows_f.at[pl.ds(b * CHUNK, CHUNK)]

        def w_buf(b):
            return w_v.at[pl.ds(b * CHUNK, CHUNK)]

        def start_gather(ch, b):
            @pl.when(cid == 0)
            def _():
                pltpu.async_copy(h0_hbm.at[src_v.at[ch]], rows_buf(b), gsem)

            @pl.when(cid == 1)
            def _():
                pltpu.async_copy(h1_hbm.at[src_v.at[ch]], rows_buf(b), gsem)

        def wait_gather(b):
            pltpu.make_async_copy(h0_hbm.at[src_v.at[0]], rows_buf(b), gsem).wait()

        def wait_scatter(b):
            pltpu.make_async_copy(rowsf_buf(0), acc_sh.at[dst_v.at[0]], ssem).wait()
            pltpu.make_async_copy(w_buf(b), den_sh.at[dst_v.at[0]], dsem).wait()

        hc = n_chunks // 2

        for p in range(2):
            pltpu.sync_copy(src_hbm.at[sid, pl.ds(p * hc, hc)], src_v)
            pltpu.sync_copy(dst_hbm.at[sid, pl.ds(p * hc, hc)], dst_v)
            base_p = base + p * hc * CHUNK
            start_gather(0, 0)
            start_gather(1, 1)

            def chunk_body(ch, carry, base_p=base_p):
                b = lax.rem(ch, NB)
                pb = lax.rem(ch + 2, NB)
                wb = b * CHUNK

                @pl.when(ch >= 2)
                def _():
                    wait_scatter(pb)

                @pl.when(ch + 2 < hc)
                def _():
                    start_gather(ch + 2, pb)

                for j in range(8):
                    sv = src_v[ch, pl.ds(j * 16, 16)]
                    dv = dst_v[ch, pl.ds(j * 16, 16)]
                    a = plsc.load_gather(as_v, [sv]) + plsc.load_gather(ad_v, [dv])
                    a = jnp.where(a > 0, a, NEG * a)
                    w = jnp.exp(a - vc)
                    eid = base_p + ch * CHUNK + j * 16 + iota16
                    w = jnp.where(eid < e_tot, w, 0.0)
                    w_v[pl.ds(wb + j * 16, 16)] = w

                pltpu.async_copy(w_buf(b), den_sh.at[dst_v.at[ch]], dsem,
                                 add=True)

                wait_gather(b)
                parity = lax.rem(ch, 2)
                fb = parity * CHUNK

                @plsc.parallel_loop(0, CHUNK, unroll=4)
                def _(e):
                    we = plsc.load_gather(w_v, [jnp.broadcast_to(wb + e, (16,))])
                    r = wb + e
                    rf = fb + e
                    for g in range(DH // 32):
                        v32 = rows_bf[r, pl.ds(g * 32, 32)]
                        lo, hi = plsc.unpack(
                            v32, format=plsc.PackFormat.INTERLEAVED,
                            preferred_element_type=jnp.float32)
                        rows_f[rf, pl.ds(g * 32, 16)] = lo * we
                        rows_f[rf, pl.ds(g * 32 + 16, 16)] = hi * we

                pltpu.async_copy(rowsf_buf(parity), acc_sh.at[dst_v.at[ch]],
                                 ssem, add=True)
                return carry

            lax.fori_loop(0, hc, chunk_body, 0)
            wait_scatter((hc - 2) % NB)
            wait_scatter((hc - 1) % NB)
        plsc.subcore_barrier()

        pltpu.sync_copy(den_sh.at[pl.ds(row0, stripe)], den_v)
        pltpu.sync_copy(bias_hbm, bias_v)
        col0 = cid * DH
        for i in range(stripe // 16):
            sl = pl.ds(i * 16, 16)
            den_v[sl] = 1.0 / (den_v[sl] + 1e-16)
        out_row0 = row0
        off = 0
        for sz in (CHUNK, CHUNK, CHUNK, CHUNK, CHUNK):
            pltpu.sync_copy(acc_sh.at[pl.ds(out_row0 + off, sz)],
                            rows_f.at[pl.ds(0, sz)])
            _off = off

            @plsc.parallel_loop(0, sz, unroll=4)
            def _(r):
                d = plsc.load_gather(
                    den_v, [jnp.broadcast_to(_off + r, (16,))])
                for k in range(DH // 16):
                    sl = pl.ds(k * 16, 16)
                    bv = bias_v[pl.ds(col0 + k * 16, 16)]
                    rows_f[r, sl] = jnp.maximum(rows_f[r, sl] * d + bv, 0.0)
            pltpu.sync_copy(rows_f.at[pl.ds(0, sz)],
                            out_hbm.at[pl.ds(out_row0 + off, sz),
                                       pl.ds(col0, DH)])
            off += sz

    return sc_edge


def _interleave_bf16(hh):
    n = hh.shape[0]
    t = hh.reshape(n, 2, 2, 16).transpose(0, 1, 3, 2).reshape(n, 64)
    return t.astype(jnp.bfloat16)


@jax.jit
def kernel(x, edge_index, W, att_src, att_dst, bias):
    n = x.shape[0]
    e = edge_index.shape[1]
    e_tot = e + n

    h0, h1, als, ald = _tc_prep(x, W, att_src, att_dst)
    a_src = als.reshape(n)
    a_dst = ald.reshape(n)

    m = jnp.max(a_src) + jnp.max(a_dst)
    c = jnp.where(m > 0, m, NEG * m)
    c_vec = jnp.full((16,), c, jnp.float32)

    ei = edge_index.astype(jnp.int32)
    loops = jnp.arange(n, dtype=jnp.int32)
    src = jnp.concatenate([ei[0], loops])
    dst = jnp.concatenate([ei[1], loops])
    n_sub = 16
    per_round = n_sub * CHUNK
    n_chunks = (e_tot + per_round - 1) // per_round
    e_pad = n_chunks * per_round
    src = jnp.pad(src, (0, e_pad - e_tot)).reshape(n_sub, n_chunks, CHUNK)
    dst = jnp.pad(dst, (0, e_pad - e_tot)).reshape(n_sub, n_chunks, CHUNK)

    sc_edge = _make_sc_edge(n_chunks, e_tot)
    (out,) = sc_edge(_interleave_bf16(h0), _interleave_bf16(h1),
                     a_src, a_dst, c_vec, src, dst, bias)
    return out[:n]
